# baseline (speedup 1.0000x reference)
import functools

import jax
import jax.numpy as jnp
from jax import lax
from jax.experimental import pallas as pl
from jax.experimental.pallas import tpu as pltpu
from jax.experimental.pallas import tpu_sc as plsc

_N = 10000
_EMB = 32
_ANCH = 32
_HEADS = 4
_DH = 8
_E0 = 640000
_ADD = int(_E0 * 0.01)
_ETOT = 2 * _ADD + _N + _E0
_L = 16
_NW = 32
_NSUB = 16
_CHUNK = 128
_NCH = -(-_ETOT // (_NW * _CHUNK))
_PER_TILE = _NCH * _CHUNK
_EPAD = _NW * _PER_TILE
_NPAD = 10048
_STRIPE = _NPAD // _NSUB
_SFLAT = _STRIPE * _HEADS
_NFLAT = _NPAD * _HEADS


def _qk_body(emb_ref, dst_ref, se_ref, w1_ref, w2_ref, bh_ref, qt_ref,
             kt_ref, q_ref, k_ref):
    f32 = jnp.float32
    sw = jnp.dot(se_ref[...], w1_ref[...], preferred_element_type=f32)
    pos = (jnp.dot(dst_ref[...], sw, preferred_element_type=f32) * (1.0 / _ANCH)
           + jnp.dot(emb_ref[...], w2_ref[...], preferred_element_type=f32)
           + bh_ref[...])
    q_ref[...] = jnp.dot(pos, qt_ref[...], preferred_element_type=f32)
    k_ref[...] = jnp.dot(pos, kt_ref[...], preferred_element_type=f32)


_mesh = plsc.VectorSubcoreMesh(core_axis_name="c", subcore_axis_name="s")
_sc_params = pltpu.CompilerParams(
    needs_layout_passes=False, use_tc_tiling_on_sc=False)


def _edge_attention_body(q_hbm, k_hbm, rows_hbm, cols_hbm, z_hbm,
                         exp_hbm, na_hbm, nb_hbm,
                         rv, cv, qv, kv, evals, nrm, sbuf, sacc, nsh,
                         sem1, sem2):
    c = lax.axis_index("c")
    s = lax.axis_index("s")
    wid = s * 2 + c
    pltpu.sync_copy(z_hbm, nrm)
    base = wid * _PER_TILE

    def chunk_body(i, carry):
        off = base + i * _CHUNK
        pltpu.sync_copy(rows_hbm.at[pl.ds(off, _CHUNK)], rv)
        pltpu.sync_copy(cols_hbm.at[pl.ds(off, _CHUNK)], cv)
        cp1 = pltpu.async_copy(q_hbm.at[rv], qv, sem1)
        cp2 = pltpu.async_copy(k_hbm.at[cv], kv, sem2)
        cp1.wait()
        cp2.wait()
        for g in range(_CHUNK // _L):
            ei = lax.iota(jnp.int32, _L) + (g * _L)
            rl4 = rv[pl.ds(g * _L, _L)] * _HEADS
            for h in range(_HEADS):
                acc = None
                for d in range(_DH):
                    ci = jnp.full((_L,), h * _DH + d, jnp.int32)
                    qc = plsc.load_gather(qv, [ei, ci])
                    kc = plsc.load_gather(kv, [ei, ci])
                    acc = qc * kc if acc is None else acc + qc * kc
                att = jnp.minimum(jnp.maximum(acc, -10.0), 10.0)
                ex = jnp.exp(att)
                hs = jnp.full((_L,), h, jnp.int32)
                plsc.store_scatter(evals, [ei, hs], ex)
                plsc.addupdate_scatter(nrm, [rl4 + h], ex)
        pltpu.sync_copy(evals, exp_hbm.at[pl.ds(off, _CHUNK)])
        return carry

    lax.fori_loop(0, _NCH, chunk_body, 0)
    pltpu.sync_copy(nrm, nsh.at[s])
    plsc.subcore_barrier()
    sl = s * _SFLAT
    pltpu.sync_copy(nsh.at[0, pl.ds(sl, _SFLAT)], sacc)

    def slot_body(t, carry):
        pltpu.sync_copy(nsh.at[t, pl.ds(sl, _SFLAT)], sbuf)
        for j in range(_SFLAT // _L):
            ix = pl.ds(j * _L, _L)
            sacc[ix] = sacc[ix] + sbuf[ix]
        return carry

    lax.fori_loop(1, _NSUB, slot_body, 0)

    @pl.when(c == 0)
    def _():
        pltpu.sync_copy(sacc, na_hbm.at[pl.ds(sl, _SFLAT)])

    @pl.when(c == 1)
    def _():
        pltpu.sync_copy(sacc, nb_hbm.at[pl.ds(sl, _SFLAT)])


_edge_attention = functools.partial(
    pl.kernel,
    out_type=[
        jax.ShapeDtypeStruct((_EPAD, _HEADS), jnp.float32),
        jax.ShapeDtypeStruct((_NFLAT,), jnp.float32),
        jax.ShapeDtypeStruct((_NFLAT,), jnp.float32),
    ],
    scratch_types=[
        pltpu.VMEM((_CHUNK,), jnp.int32),
        pltpu.VMEM((_CHUNK,), jnp.int32),
        pltpu.VMEM((_CHUNK, _EMB), jnp.float32),
        pltpu.VMEM((_CHUNK, _EMB), jnp.float32),
        pltpu.VMEM((_CHUNK, _HEADS), jnp.float32),
        pltpu.VMEM((_NFLAT,), jnp.float32),
        pltpu.VMEM((_SFLAT,), jnp.float32),
        pltpu.VMEM((_SFLAT,), jnp.float32),
        pltpu.VMEM_SHARED((_NSUB, _NFLAT), jnp.float32),
        pltpu.SemaphoreType.DMA,
        pltpu.SemaphoreType.DMA,
    ],
    mesh=_mesh,
    compiler_params=_sc_params,
)(_edge_attention_body)


def _normalize_body(rows_hbm, exp_hbm, na_hbm, nb_hbm, out_hbm,
                    rv, ev, nav, nbv, av, sem1, sem2):
    c = lax.axis_index("c")
    s = lax.axis_index("s")
    wid = s * 2 + c
    base = wid * _PER_TILE

    def chunk_body(i, carry):
        off = base + i * _CHUNK
        pltpu.sync_copy(rows_hbm.at[pl.ds(off, _CHUNK)], rv)
        pltpu.sync_copy(exp_hbm.at[pl.ds(off, _CHUNK)], ev)
        cp1 = pltpu.async_copy(na_hbm.at[rv], nav, sem1)
        cp2 = pltpu.async_copy(nb_hbm.at[rv], nbv, sem2)
        cp1.wait()
        cp2.wait()
        for g in range(_CHUNK // _L):
            ei = lax.iota(jnp.int32, _L) + (g * _L)
            acc = jnp.zeros((_L,), jnp.float32)
            for h in range(_HEADS):
                hs = jnp.full((_L,), h, jnp.int32)
                eh = plsc.load_gather(ev, [ei, hs])
                nh = plsc.load_gather(nav, [ei, hs]) + plsc.load_gather(nbv, [ei, hs])
                acc = acc + eh / (nh + 1e-8)
            av[pl.ds(g * _L, _L)] = acc
        pltpu.sync_copy(av, out_hbm.at[pl.ds(off, _CHUNK)])
        return carry

    lax.fori_loop(0, _NCH, chunk_body, 0)


_normalize = functools.partial(
    pl.kernel,
    out_type=jax.ShapeDtypeStruct((_EPAD,), jnp.float32),
    scratch_types=[
        pltpu.VMEM((_CHUNK,), jnp.int32),
        pltpu.VMEM((_CHUNK, _HEADS), jnp.float32),
        pltpu.VMEM((_CHUNK, _HEADS), jnp.float32),
        pltpu.VMEM((_CHUNK, _HEADS), jnp.float32),
        pltpu.VMEM((_CHUNK,), jnp.float32),
        pltpu.SemaphoreType.DMA,
        pltpu.SemaphoreType.DMA,
    ],
    mesh=_mesh,
    compiler_params=_sc_params,
)(_normalize_body)


def kernel(embeds, edge_index, anchorset_id, dists_array, Wh, bh, qTrans,
           kTrans, vTrans):
    del vTrans
    f32 = jnp.float32
    set_emb = jnp.take(embeds, anchorset_id, axis=0)
    w1 = Wh[:_EMB]
    w2 = Wh[_EMB:]
    emb_p = jnp.pad(embeds, ((0, _NPAD - _N), (0, 0)))
    dst_p = jnp.pad(dists_array, ((0, _NPAD - _N), (0, 0)))
    q_tab, k_tab = pl.pallas_call(
        _qk_body,
        out_shape=[jax.ShapeDtypeStruct((_NPAD, _EMB), f32)] * 2,
    )(emb_p, dst_p, set_emb, w1, w2, bh.reshape(1, _EMB), qTrans, kTrans)

    rows = edge_index[0]
    cols = edge_index[1]
    ka, kb = jax.random.split(jax.random.key(1))
    tr = rows[jax.random.randint(ka, (_ADD,), 0, _E0)]
    tc = cols[jax.random.randint(kb, (_ADD,), 0, _E0)]
    loop = jnp.arange(_N, dtype=rows.dtype)
    new_rows = jnp.concatenate([tr, tc, loop, rows])
    new_cols = jnp.concatenate([tc, tr, loop, cols])
    rows_p = jnp.pad(new_rows, (0, _EPAD - _ETOT), constant_values=_N)
    cols_p = jnp.pad(new_cols, (0, _EPAD - _ETOT), constant_values=_N)
    z = jnp.zeros((_NFLAT,), f32)

    exp_e, na, nb = _edge_attention(q_tab, k_tab, rows_p, cols_p, z)
    att = _normalize(rows_p, exp_e,
                     na.reshape(_NPAD, _HEADS), nb.reshape(_NPAD, _HEADS))
    return att[:_ETOT], new_rows, new_cols


# speedup vs baseline: 1.3298x; 1.3298x over previous
import functools

import jax
import jax.numpy as jnp
from jax import lax
from jax.experimental import pallas as pl
from jax.experimental.pallas import tpu as pltpu
from jax.experimental.pallas import tpu_sc as plsc

_N = 10000
_EMB = 32
_ANCH = 32
_HEADS = 4
_DH = 8
_E0 = 640000
_ADD = int(_E0 * 0.01)
_ETOT = 2 * _ADD + _N + _E0
_L = 16
_NW = 32
_CHUNK = 128
_NCH = -(-_ETOT // (_NW * _CHUNK))
_PER_TILE = _NCH * _CHUNK
_EPAD = _NW * _PER_TILE
_NPAD = _N + 8
_HPAD = 8


def _qk_body(emb_ref, dst_ref, se_ref, w1_ref, w2_ref, bh_ref, qt_ref,
             kt_ref, q_ref, k_ref):
    f32 = jnp.float32
    sw = jnp.dot(se_ref[...], w1_ref[...], preferred_element_type=f32)
    pos = (jnp.dot(dst_ref[...], sw, preferred_element_type=f32) * (1.0 / _ANCH)
           + jnp.dot(emb_ref[...], w2_ref[...], preferred_element_type=f32)
           + bh_ref[...])
    q_ref[...] = jnp.dot(pos, qt_ref[...], preferred_element_type=f32)
    k_ref[...] = jnp.dot(pos, kt_ref[...], preferred_element_type=f32)


_mesh = plsc.VectorSubcoreMesh(core_axis_name="c", subcore_axis_name="s")
_sc_params = pltpu.CompilerParams(
    needs_layout_passes=False, use_tc_tiling_on_sc=False)


def _edge_attention_body(q_hbm, k_hbm, rows_hbm, cols_hbm, z_hbm,
                         exp_hbm, na_hbm, nb_hbm,
                         rv0, cv0, qv0, kv0, ev0,
                         rv1, cv1, qv1, kv1, ev1,
                         vals, nsh,
                         si0, si1, sq0, sk0, sq1, sk1):
    c = lax.axis_index("c")
    s = lax.axis_index("s")
    wid = s * 2 + c
    rv = (rv0, rv1)
    cv = (cv0, cv1)
    qv = (qv0, qv1)
    kv = (kv0, kv1)
    ev = (ev0, ev1)
    si = (si0, si1)
    sq = (sq0, sq1)
    sk = (sk0, sk1)
    base = wid * _PER_TILE
    pltpu.sync_copy(z_hbm.at[pl.ds(0, _CHUNK)], vals)

    @pl.when(s == 0)
    def _():
        pltpu.sync_copy(z_hbm, nsh)

    plsc.subcore_barrier()

    def issue_idx(i, b):
        off = base + i * _CHUNK
        pltpu.async_copy(rows_hbm.at[pl.ds(off, _CHUNK)], rv[b], si[b])
        pltpu.async_copy(cols_hbm.at[pl.ds(off, _CHUNK)], cv[b], si[b])

    def wait_idx(i, b):
        off = base + i * _CHUNK
        pltpu.make_async_copy(rows_hbm.at[pl.ds(off, _CHUNK)], rv[b], si[b]).wait()
        pltpu.make_async_copy(cols_hbm.at[pl.ds(off, _CHUNK)], cv[b], si[b]).wait()

    def issue_gather(b):
        pltpu.async_copy(q_hbm.at[rv[b]], qv[b], sq[b])
        pltpu.async_copy(k_hbm.at[cv[b]], kv[b], sk[b])

    def wait_gather(b):
        pltpu.make_async_copy(q_hbm.at[rv[b]], qv[b], sq[b]).wait()
        pltpu.make_async_copy(k_hbm.at[cv[b]], kv[b], sk[b]).wait()

    def process(i, b):
        for g in range(_CHUNK // _L):
            ei = lax.iota(jnp.int32, _L) + (g * _L)
            for h in range(_HEADS):
                acc = None
                for d in range(_DH):
                    ci = jnp.full((_L,), h * _DH + d, jnp.int32)
                    qc = plsc.load_gather(qv[b], [ei, ci])
                    kc = plsc.load_gather(kv[b], [ei, ci])
                    acc = qc * kc if acc is None else acc + qc * kc
                att = jnp.minimum(jnp.maximum(acc, -10.0), 10.0)
                ex = jnp.exp(att)
                hs = jnp.full((_L,), h, jnp.int32)
                plsc.store_scatter(vals, [ei, hs], ex)
                plsc.store_scatter(ev[b], [ei, hs], ex)
        off = base + i * _CHUNK
        pltpu.sync_copy(ev[b], exp_hbm.at[pl.ds(off, _CHUNK)])
        pltpu.sync_copy(vals, nsh.at[rv[b]], add=True)

    issue_idx(0, 0)
    issue_idx(1, 1)
    wait_idx(0, 0)
    issue_gather(0)

    def pair_body(k2, carry):
        for b in (0, 1):
            i = k2 * 2 + b
            nb = 1 - b

            @pl.when(i + 1 < _NCH)
            def _():
                wait_idx(i + 1, nb)
                issue_gather(nb)

            wait_gather(b)
            process(i, b)

            @pl.when(i + 2 < _NCH)
            def _():
                issue_idx(i + 2, b)

        return carry

    lax.fori_loop(0, (_NCH + 1) // 2, pair_body, 0)
    plsc.subcore_barrier()

    @pl.when(jnp.logical_and(s == 0, c == 0))
    def _():
        pltpu.sync_copy(nsh, na_hbm)

    @pl.when(jnp.logical_and(s == 0, c == 1))
    def _():
        pltpu.sync_copy(nsh, nb_hbm)


_edge_attention = functools.partial(
    pl.kernel,
    out_type=[
        jax.ShapeDtypeStruct((_EPAD, _HEADS), jnp.float32),
        jax.ShapeDtypeStruct((_NPAD, _HPAD), jnp.float32),
        jax.ShapeDtypeStruct((_NPAD, _HPAD), jnp.float32),
    ],
    scratch_types=[
        pltpu.VMEM((_CHUNK,), jnp.int32),
        pltpu.VMEM((_CHUNK,), jnp.int32),
        pltpu.VMEM((_CHUNK, _EMB), jnp.float32),
        pltpu.VMEM((_CHUNK, _EMB), jnp.float32),
        pltpu.VMEM((_CHUNK, _HEADS), jnp.float32),
        pltpu.VMEM((_CHUNK,), jnp.int32),
        pltpu.VMEM((_CHUNK,), jnp.int32),
        pltpu.VMEM((_CHUNK, _EMB), jnp.float32),
        pltpu.VMEM((_CHUNK, _EMB), jnp.float32),
        pltpu.VMEM((_CHUNK, _HEADS), jnp.float32),
        pltpu.VMEM((_CHUNK, _HPAD), jnp.float32),
        pltpu.VMEM_SHARED((_NPAD, _HPAD), jnp.float32),
        pltpu.SemaphoreType.DMA,
        pltpu.SemaphoreType.DMA,
        pltpu.SemaphoreType.DMA,
        pltpu.SemaphoreType.DMA,
        pltpu.SemaphoreType.DMA,
        pltpu.SemaphoreType.DMA,
    ],
    mesh=_mesh,
    compiler_params=_sc_params,
)(_edge_attention_body)


def _normalize_body(rows_hbm, exp_hbm, na_hbm, nb_hbm, out_hbm,
                    rv0, ev0, na0, nb0, rv1, ev1, na1, nb1, av,
                    si0, si1, sa0, sb0, sa1, sb1):
    c = lax.axis_index("c")
    s = lax.axis_index("s")
    wid = s * 2 + c
    rv = (rv0, rv1)
    ev = (ev0, ev1)
    nav = (na0, na1)
    nbv = (nb0, nb1)
    si = (si0, si1)
    sa = (sa0, sa1)
    sb = (sb0, sb1)
    base = wid * _PER_TILE

    def issue_idx(i, b):
        off = base + i * _CHUNK
        pltpu.async_copy(rows_hbm.at[pl.ds(off, _CHUNK)], rv[b], si[b])
        pltpu.async_copy(exp_hbm.at[pl.ds(off, _CHUNK)], ev[b], si[b])

    def wait_idx(i, b):
        off = base + i * _CHUNK
        pltpu.make_async_copy(rows_hbm.at[pl.ds(off, _CHUNK)], rv[b], si[b]).wait()
        pltpu.make_async_copy(exp_hbm.at[pl.ds(off, _CHUNK)], ev[b], si[b]).wait()

    def issue_gather(b):
        pltpu.async_copy(na_hbm.at[rv[b]], nav[b], sa[b])
        pltpu.async_copy(nb_hbm.at[rv[b]], nbv[b], sb[b])

    def wait_gather(b):
        pltpu.make_async_copy(na_hbm.at[rv[b]], nav[b], sa[b]).wait()
        pltpu.make_async_copy(nb_hbm.at[rv[b]], nbv[b], sb[b]).wait()

    def process(i, b):
        for g in range(_CHUNK // _L):
            ei = lax.iota(jnp.int32, _L) + (g * _L)
            acc = jnp.zeros((_L,), jnp.float32)
            for h in range(_HEADS):
                hs = jnp.full((_L,), h, jnp.int32)
                eh = plsc.load_gather(ev[b], [ei, hs])
                nh = (plsc.load_gather(nav[b], [ei, hs])
                      + plsc.load_gather(nbv[b], [ei, hs]))
                acc = acc + eh / (nh + 1e-8)
            av[pl.ds(g * _L, _L)] = acc
        off = base + i * _CHUNK
        pltpu.sync_copy(av, out_hbm.at[pl.ds(off, _CHUNK)])

    issue_idx(0, 0)
    issue_idx(1, 1)
    wait_idx(0, 0)
    issue_gather(0)

    def pair_body(k2, carry):
        for b in (0, 1):
            i = k2 * 2 + b
            nb = 1 - b

            @pl.when(i + 1 < _NCH)
            def _():
                wait_idx(i + 1, nb)
                issue_gather(nb)

            wait_gather(b)
            process(i, b)

            @pl.when(i + 2 < _NCH)
            def _():
                issue_idx(i + 2, b)

        return carry

    lax.fori_loop(0, (_NCH + 1) // 2, pair_body, 0)


_normalize = functools.partial(
    pl.kernel,
    out_type=jax.ShapeDtypeStruct((_EPAD,), jnp.float32),
    scratch_types=[
        pltpu.VMEM((_CHUNK,), jnp.int32),
        pltpu.VMEM((_CHUNK, _HEADS), jnp.float32),
        pltpu.VMEM((_CHUNK, _HPAD), jnp.float32),
        pltpu.VMEM((_CHUNK, _HPAD), jnp.float32),
        pltpu.VMEM((_CHUNK,), jnp.int32),
        pltpu.VMEM((_CHUNK, _HEADS), jnp.float32),
        pltpu.VMEM((_CHUNK, _HPAD), jnp.float32),
        pltpu.VMEM((_CHUNK, _HPAD), jnp.float32),
        pltpu.VMEM((_CHUNK,), jnp.float32),
        pltpu.SemaphoreType.DMA,
        pltpu.SemaphoreType.DMA,
        pltpu.SemaphoreType.DMA,
        pltpu.SemaphoreType.DMA,
        pltpu.SemaphoreType.DMA,
        pltpu.SemaphoreType.DMA,
    ],
    mesh=_mesh,
    compiler_params=_sc_params,
)(_normalize_body)


def kernel(embeds, edge_index, anchorset_id, dists_array, Wh, bh, qTrans,
           kTrans, vTrans):
    del vTrans
    f32 = jnp.float32
    set_emb = jnp.take(embeds, anchorset_id, axis=0)
    w1 = Wh[:_EMB]
    w2 = Wh[_EMB:]
    emb_p = jnp.pad(embeds, ((0, _NPAD - _N), (0, 0)))
    dst_p = jnp.pad(dists_array, ((0, _NPAD - _N), (0, 0)))
    q_tab, k_tab = pl.pallas_call(
        _qk_body,
        out_shape=[jax.ShapeDtypeStruct((_NPAD, _EMB), f32)] * 2,
    )(emb_p, dst_p, set_emb, w1, w2, bh.reshape(1, _EMB), qTrans, kTrans)

    rows = edge_index[0]
    cols = edge_index[1]
    ka, kb = jax.random.split(jax.random.key(1))
    tr = rows[jax.random.randint(ka, (_ADD,), 0, _E0)]
    tc = cols[jax.random.randint(kb, (_ADD,), 0, _E0)]
    loop = jnp.arange(_N, dtype=rows.dtype)
    new_rows = jnp.concatenate([tr, tc, loop, rows])
    new_cols = jnp.concatenate([tc, tr, loop, cols])
    rows_p = jnp.pad(new_rows, (0, _EPAD - _ETOT), constant_values=_N)
    cols_p = jnp.pad(new_cols, (0, _EPAD - _ETOT), constant_values=_N)
    z = jnp.zeros((_NPAD, _HPAD), f32)

    exp_e, na, nb = _edge_attention(q_tab, k_tab, rows_p, cols_p, z)
    att = _normalize(rows_p, exp_e, na, nb)
    return att[:_ETOT], new_rows, new_cols


# speedup vs baseline: 1.3707x; 1.0307x over previous
import functools

import jax
import jax.numpy as jnp
from jax import lax
from jax.experimental import pallas as pl
from jax.experimental.pallas import tpu as pltpu
from jax.experimental.pallas import tpu_sc as plsc

_N = 10000
_EMB = 32
_ANCH = 32
_HEADS = 4
_DH = 8
_E0 = 640000
_ADD = int(_E0 * 0.01)
_ETOT = 2 * _ADD + _N + _E0
_L = 16
_NW = 32
_CHUNK = 128
_NCH = -(-_ETOT // (_NW * _CHUNK))
_PER_TILE = _NCH * _CHUNK
_EPAD = _NW * _PER_TILE
_NPAD = _N + 8
_HPAD = 8


def _qk_body(emb_ref, dst_ref, se_ref, w1_ref, w2_ref, bh_ref, qt_ref,
             kt_ref, q_ref, k_ref):
    f32 = jnp.float32
    sw = jnp.dot(se_ref[...], w1_ref[...], preferred_element_type=f32)
    pos = (jnp.dot(dst_ref[...], sw, preferred_element_type=f32) * (1.0 / _ANCH)
           + jnp.dot(emb_ref[...], w2_ref[...], preferred_element_type=f32)
           + bh_ref[...])
    q_ref[...] = jnp.dot(pos, qt_ref[...], preferred_element_type=f32)
    k_ref[...] = jnp.dot(pos, kt_ref[...], preferred_element_type=f32)


_mesh = plsc.VectorSubcoreMesh(core_axis_name="c", subcore_axis_name="s")
_sc_params = pltpu.CompilerParams(
    needs_layout_passes=False, use_tc_tiling_on_sc=False)


def _edge_attention_body(q_hbm, k_hbm, rows_hbm, cols_hbm, z_hbm,
                         exp_hbm, na_hbm, nb_hbm,
                         rv0, cv0, qv0, kv0, ev0, vals0, rs0,
                         rv1, cv1, qv1, kv1, ev1, vals1, rs1,
                         nsh,
                         si0, si1, sq0, sk0, sq1, sk1,
                         sew0, sew1, sad0, sad1):
    c = lax.axis_index("c")
    s = lax.axis_index("s")
    wid = s * 2 + c
    rv = (rv0, rv1)
    cv = (cv0, cv1)
    qv = (qv0, qv1)
    kv = (kv0, kv1)
    ev = (ev0, ev1)
    vals = (vals0, vals1)
    rs = (rs0, rs1)
    si = (si0, si1)
    sq = (sq0, sq1)
    sk = (sk0, sk1)
    sew = (sew0, sew1)
    sad = (sad0, sad1)
    base = wid * _PER_TILE
    pltpu.sync_copy(z_hbm.at[pl.ds(0, _CHUNK)], vals0)
    pltpu.sync_copy(z_hbm.at[pl.ds(0, _CHUNK)], vals1)

    @pl.when(s == 0)
    def _():
        pltpu.sync_copy(z_hbm, nsh)

    plsc.subcore_barrier()

    def issue_idx(i, b):
        off = base + i * _CHUNK
        pltpu.async_copy(rows_hbm.at[pl.ds(off, _CHUNK)], rv[b], si[b])
        pltpu.async_copy(cols_hbm.at[pl.ds(off, _CHUNK)], cv[b], si[b])

    def wait_idx(i, b):
        off = base + i * _CHUNK
        pltpu.make_async_copy(rows_hbm.at[pl.ds(off, _CHUNK)], rv[b], si[b]).wait()
        pltpu.make_async_copy(cols_hbm.at[pl.ds(off, _CHUNK)], cv[b], si[b]).wait()

    def issue_gather(b):
        pltpu.async_copy(q_hbm.at[rv[b]], qv[b], sq[b])
        pltpu.async_copy(k_hbm.at[cv[b]], kv[b], sk[b])

    def wait_gather(b):
        pltpu.make_async_copy(q_hbm.at[rv[b]], qv[b], sq[b]).wait()
        pltpu.make_async_copy(k_hbm.at[cv[b]], kv[b], sk[b]).wait()

    def wait_write(i, b):
        off = base + i * _CHUNK
        pltpu.make_async_copy(ev[b], exp_hbm.at[pl.ds(off, _CHUNK)], sew[b]).wait()
        pltpu.make_async_copy(vals[b], nsh.at[rs[b]], sad[b]).wait()

    def process(i, b):
        @pl.when(i >= 2)
        def _():
            wait_write(i - 2, b)

        for g in range(_CHUNK // _L):
            ei = lax.iota(jnp.int32, _L) + (g * _L)
            for h in range(_HEADS):
                acc = None
                for d in range(_DH):
                    ci = jnp.full((_L,), h * _DH + d, jnp.int32)
                    qc = plsc.load_gather(qv[b], [ei, ci])
                    kc = plsc.load_gather(kv[b], [ei, ci])
                    acc = qc * kc if acc is None else acc + qc * kc
                att = jnp.minimum(jnp.maximum(acc, -10.0), 10.0)
                ex = jnp.exp(att)
                hs = jnp.full((_L,), h, jnp.int32)
                plsc.store_scatter(vals[b], [ei, hs], ex)
                plsc.store_scatter(ev[b], [ei, hs], ex)
        for j in range(_CHUNK // _L):
            ix = pl.ds(j * _L, _L)
            rs[b][ix] = rv[b][ix]
        off = base + i * _CHUNK
        pltpu.async_copy(ev[b], exp_hbm.at[pl.ds(off, _CHUNK)], sew[b])
        pltpu.async_copy(vals[b], nsh.at[rs[b]], sad[b], add=True)

    issue_idx(0, 0)
    issue_idx(1, 1)
    wait_idx(0, 0)
    issue_gather(0)

    def pair_body(k2, carry):
        for b in (0, 1):
            i = k2 * 2 + b
            nb = 1 - b

            @pl.when(i + 1 < _NCH)
            def _():
                wait_idx(i + 1, nb)
                issue_gather(nb)

            wait_gather(b)
            process(i, b)

            @pl.when(i + 2 < _NCH)
            def _():
                issue_idx(i + 2, b)

        return carry

    lax.fori_loop(0, (_NCH + 1) // 2, pair_body, 0)
    wait_write(_NCH - 2, (_NCH - 2) % 2)
    wait_write(_NCH - 1, (_NCH - 1) % 2)
    plsc.subcore_barrier()

    @pl.when(jnp.logical_and(s == 0, c == 0))
    def _():
        pltpu.sync_copy(nsh, na_hbm)

    @pl.when(jnp.logical_and(s == 0, c == 1))
    def _():
        pltpu.sync_copy(nsh, nb_hbm)


_edge_attention = functools.partial(
    pl.kernel,
    out_type=[
        jax.ShapeDtypeStruct((_EPAD, _HEADS), jnp.float32),
        jax.ShapeDtypeStruct((_NPAD, _HPAD), jnp.float32),
        jax.ShapeDtypeStruct((_NPAD, _HPAD), jnp.float32),
    ],
    scratch_types=[
        pltpu.VMEM((_CHUNK,), jnp.int32),
        pltpu.VMEM((_CHUNK,), jnp.int32),
        pltpu.VMEM((_CHUNK, _EMB), jnp.float32),
        pltpu.VMEM((_CHUNK, _EMB), jnp.float32),
        pltpu.VMEM((_CHUNK, _HEADS), jnp.float32),
        pltpu.VMEM((_CHUNK, _HPAD), jnp.float32),
        pltpu.VMEM((_CHUNK,), jnp.int32),
        pltpu.VMEM((_CHUNK,), jnp.int32),
        pltpu.VMEM((_CHUNK,), jnp.int32),
        pltpu.VMEM((_CHUNK, _EMB), jnp.float32),
        pltpu.VMEM((_CHUNK, _EMB), jnp.float32),
        pltpu.VMEM((_CHUNK, _HEADS), jnp.float32),
        pltpu.VMEM((_CHUNK, _HPAD), jnp.float32),
        pltpu.VMEM((_CHUNK,), jnp.int32),
        pltpu.VMEM_SHARED((_NPAD, _HPAD), jnp.float32),
        pltpu.SemaphoreType.DMA,
        pltpu.SemaphoreType.DMA,
        pltpu.SemaphoreType.DMA,
        pltpu.SemaphoreType.DMA,
        pltpu.SemaphoreType.DMA,
        pltpu.SemaphoreType.DMA,
        pltpu.SemaphoreType.DMA,
        pltpu.SemaphoreType.DMA,
        pltpu.SemaphoreType.DMA,
        pltpu.SemaphoreType.DMA,
    ],
    mesh=_mesh,
    compiler_params=_sc_params,
)(_edge_attention_body)


def _normalize_body(rows_hbm, exp_hbm, na_hbm, nb_hbm, out_hbm,
                    rv0, ev0, na0, nb0, av0, rv1, ev1, na1, nb1, av1,
                    si0, si1, sa0, sb0, sa1, sb1, sw0, sw1):
    c = lax.axis_index("c")
    s = lax.axis_index("s")
    wid = s * 2 + c
    rv = (rv0, rv1)
    ev = (ev0, ev1)
    nav = (na0, na1)
    nbv = (nb0, nb1)
    av = (av0, av1)
    si = (si0, si1)
    sa = (sa0, sa1)
    sb = (sb0, sb1)
    sw = (sw0, sw1)
    base = wid * _PER_TILE

    def issue_idx(i, b):
        off = base + i * _CHUNK
        pltpu.async_copy(rows_hbm.at[pl.ds(off, _CHUNK)], rv[b], si[b])
        pltpu.async_copy(exp_hbm.at[pl.ds(off, _CHUNK)], ev[b], si[b])

    def wait_idx(i, b):
        off = base + i * _CHUNK
        pltpu.make_async_copy(rows_hbm.at[pl.ds(off, _CHUNK)], rv[b], si[b]).wait()
        pltpu.make_async_copy(exp_hbm.at[pl.ds(off, _CHUNK)], ev[b], si[b]).wait()

    def issue_gather(b):
        pltpu.async_copy(na_hbm.at[rv[b]], nav[b], sa[b])
        pltpu.async_copy(nb_hbm.at[rv[b]], nbv[b], sb[b])

    def wait_gather(b):
        pltpu.make_async_copy(na_hbm.at[rv[b]], nav[b], sa[b]).wait()
        pltpu.make_async_copy(nb_hbm.at[rv[b]], nbv[b], sb[b]).wait()

    def wait_write(i, b):
        off = base + i * _CHUNK
        pltpu.make_async_copy(av[b], out_hbm.at[pl.ds(off, _CHUNK)], sw[b]).wait()

    def process(i, b):
        @pl.when(i >= 2)
        def _():
            wait_write(i - 2, b)

        for g in range(_CHUNK // _L):
            ei = lax.iota(jnp.int32, _L) + (g * _L)
            acc = jnp.zeros((_L,), jnp.float32)
            for h in range(_HEADS):
                hs = jnp.full((_L,), h, jnp.int32)
                eh = plsc.load_gather(ev[b], [ei, hs])
                nh = (plsc.load_gather(nav[b], [ei, hs])
                      + plsc.load_gather(nbv[b], [ei, hs]))
                acc = acc + eh / (nh + 1e-8)
            av[b][pl.ds(g * _L, _L)] = acc
        off = base + i * _CHUNK
        pltpu.async_copy(av[b], out_hbm.at[pl.ds(off, _CHUNK)], sw[b])

    issue_idx(0, 0)
    issue_idx(1, 1)
    wait_idx(0, 0)
    issue_gather(0)

    def pair_body(k2, carry):
        for b in (0, 1):
            i = k2 * 2 + b
            nb = 1 - b

            @pl.when(i + 1 < _NCH)
            def _():
                wait_idx(i + 1, nb)
                issue_gather(nb)

            wait_gather(b)
            process(i, b)

            @pl.when(i + 2 < _NCH)
            def _():
                issue_idx(i + 2, b)

        return carry

    lax.fori_loop(0, (_NCH + 1) // 2, pair_body, 0)
    wait_write(_NCH - 2, (_NCH - 2) % 2)
    wait_write(_NCH - 1, (_NCH - 1) % 2)


_normalize = functools.partial(
    pl.kernel,
    out_type=jax.ShapeDtypeStruct((_EPAD,), jnp.float32),
    scratch_types=[
        pltpu.VMEM((_CHUNK,), jnp.int32),
        pltpu.VMEM((_CHUNK, _HEADS), jnp.float32),
        pltpu.VMEM((_CHUNK, _HPAD), jnp.float32),
        pltpu.VMEM((_CHUNK, _HPAD), jnp.float32),
        pltpu.VMEM((_CHUNK,), jnp.float32),
        pltpu.VMEM((_CHUNK,), jnp.int32),
        pltpu.VMEM((_CHUNK, _HEADS), jnp.float32),
        pltpu.VMEM((_CHUNK, _HPAD), jnp.float32),
        pltpu.VMEM((_CHUNK, _HPAD), jnp.float32),
        pltpu.VMEM((_CHUNK,), jnp.float32),
        pltpu.SemaphoreType.DMA,
        pltpu.SemaphoreType.DMA,
        pltpu.SemaphoreType.DMA,
        pltpu.SemaphoreType.DMA,
        pltpu.SemaphoreType.DMA,
        pltpu.SemaphoreType.DMA,
        pltpu.SemaphoreType.DMA,
        pltpu.SemaphoreType.DMA,
    ],
    mesh=_mesh,
    compiler_params=_sc_params,
)(_normalize_body)


def kernel(embeds, edge_index, anchorset_id, dists_array, Wh, bh, qTrans,
           kTrans, vTrans):
    del vTrans
    f32 = jnp.float32
    set_emb = jnp.take(embeds, anchorset_id, axis=0)
    w1 = Wh[:_EMB]
    w2 = Wh[_EMB:]
    emb_p = jnp.pad(embeds, ((0, _NPAD - _N), (0, 0)))
    dst_p = jnp.pad(dists_array, ((0, _NPAD - _N), (0, 0)))
    q_tab, k_tab = pl.pallas_call(
        _qk_body,
        out_shape=[jax.ShapeDtypeStruct((_NPAD, _EMB), f32)] * 2,
    )(emb_p, dst_p, set_emb, w1, w2, bh.reshape(1, _EMB), qTrans, kTrans)

    rows = edge_index[0]
    cols = edge_index[1]
    ka, kb = jax.random.split(jax.random.key(1))
    tr = rows[jax.random.randint(ka, (_ADD,), 0, _E0)]
    tc = cols[jax.random.randint(kb, (_ADD,), 0, _E0)]
    loop = jnp.arange(_N, dtype=rows.dtype)
    new_rows = jnp.concatenate([tr, tc, loop, rows])
    new_cols = jnp.concatenate([tc, tr, loop, cols])
    rows_p = jnp.pad(new_rows, (0, _EPAD - _ETOT), constant_values=_N)
    cols_p = jnp.pad(new_cols, (0, _EPAD - _ETOT), constant_values=_N)
    z = jnp.zeros((_NPAD, _HPAD), f32)

    exp_e, na, nb = _edge_attention(q_tab, k_tab, rows_p, cols_p, z)
    att = _normalize(rows_p, exp_e, na, nb)
    return att[:_ETOT], new_rows, new_cols


# speedup vs baseline: 1.4403x; 1.0508x over previous
import functools

import jax
import jax.numpy as jnp
from jax import lax
from jax.experimental import pallas as pl
from jax.experimental.pallas import tpu as pltpu
from jax.experimental.pallas import tpu_sc as plsc

_N = 10000
_EMB = 32
_ANCH = 32
_HEADS = 4
_DH = 8
_E0 = 640000
_ADD = int(_E0 * 0.01)
_ETOT = 2 * _ADD + _N + _E0
_L = 16
_NW = 32
_CHUNK = 128
_NCH = -(-_ETOT // (_NW * _CHUNK))
_PER_TILE = _NCH * _CHUNK
_EPAD = _NW * _PER_TILE
_NCHT = _EPAD // _CHUNK
_NPAD = _N + 8
_HPAD = 8
_NBUF = 4


def _qk_body(emb_ref, dst_ref, se_ref, w1_ref, w2_ref, bh_ref, qt_ref,
             kt_ref, q_ref, k_ref):
    f32 = jnp.float32
    sw = jnp.dot(se_ref[...], w1_ref[...], preferred_element_type=f32)
    pos = (jnp.dot(dst_ref[...], sw, preferred_element_type=f32) * (1.0 / _ANCH)
           + jnp.dot(emb_ref[...], w2_ref[...], preferred_element_type=f32)
           + bh_ref[...])
    q_ref[...] = jnp.dot(pos, qt_ref[...], preferred_element_type=f32)
    k_ref[...] = jnp.dot(pos, kt_ref[...], preferred_element_type=f32)


_mesh = plsc.VectorSubcoreMesh(core_axis_name="c", subcore_axis_name="s")
_sc_params = pltpu.CompilerParams(
    needs_layout_passes=False, use_tc_tiling_on_sc=False)


def _edge_attention_body(q_hbm, k_hbm, rc_hbm, z_hbm,
                         exp_hbm, na_hbm, nb_hbm,
                         *refs):
    rcv = refs[0:_NBUF]
    qv = refs[_NBUF:2 * _NBUF]
    kv = refs[2 * _NBUF:3 * _NBUF]
    ev = refs[3 * _NBUF:4 * _NBUF]
    vals = refs[4 * _NBUF:5 * _NBUF]
    rs = refs[5 * _NBUF:6 * _NBUF]
    nsh = refs[6 * _NBUF]
    si = refs[6 * _NBUF + 1:7 * _NBUF + 1]
    sq = refs[7 * _NBUF + 1:8 * _NBUF + 1]
    sk = refs[8 * _NBUF + 1:9 * _NBUF + 1]
    sew = refs[9 * _NBUF + 1:10 * _NBUF + 1]
    sad = refs[10 * _NBUF + 1:11 * _NBUF + 1]
    c = lax.axis_index("c")
    s = lax.axis_index("s")
    wid = s * 2 + c
    base = wid * _PER_TILE
    chbase = wid * _NCH
    for b in range(_NBUF):
        pltpu.sync_copy(z_hbm.at[pl.ds(0, _CHUNK)], vals[b])

    @pl.when(s == 0)
    def _():
        pltpu.sync_copy(z_hbm, nsh)

    plsc.subcore_barrier()

    def issue_idx(i, b):
        pltpu.async_copy(rc_hbm.at[chbase + i], rcv[b], si[b])

    def wait_idx(i, b):
        pltpu.make_async_copy(rc_hbm.at[chbase + i], rcv[b], si[b]).wait()

    def issue_gather(b):
        pltpu.async_copy(q_hbm.at[rcv[b].at[0]], qv[b], sq[b])
        pltpu.async_copy(k_hbm.at[rcv[b].at[1]], kv[b], sk[b])

    def wait_gather(b):
        pltpu.make_async_copy(q_hbm.at[rcv[b].at[0]], qv[b], sq[b]).wait()
        pltpu.make_async_copy(k_hbm.at[rcv[b].at[1]], kv[b], sk[b]).wait()

    def wait_write(i, b):
        off = base + i * _CHUNK
        pltpu.make_async_copy(ev[b], exp_hbm.at[pl.ds(off, _CHUNK)], sew[b]).wait()
        pltpu.make_async_copy(vals[b], nsh.at[rs[b]], sad[b]).wait()

    def process(i, b):
        @pl.when(i >= _NBUF)
        def _():
            wait_write(i - _NBUF, b)

        for g in range(_CHUNK // _L):
            ei = lax.iota(jnp.int32, _L) + (g * _L)
            for h in range(_HEADS):
                acc = None
                for d in range(_DH):
                    ci = jnp.full((_L,), h * _DH + d, jnp.int32)
                    qc = plsc.load_gather(qv[b], [ei, ci])
                    kc = plsc.load_gather(kv[b], [ei, ci])
                    acc = qc * kc if acc is None else acc + qc * kc
                att = jnp.minimum(jnp.maximum(acc, -10.0), 10.0)
                ex = jnp.exp(att)
                hs = jnp.full((_L,), h, jnp.int32)
                plsc.store_scatter(vals[b], [ei, hs], ex)
                plsc.store_scatter(ev[b], [ei, hs], ex)
        for j in range(_CHUNK // _L):
            ix = pl.ds(j * _L, _L)
            rs[b][ix] = rcv[b][0, ix]
        off = base + i * _CHUNK
        pltpu.async_copy(ev[b], exp_hbm.at[pl.ds(off, _CHUNK)], sew[b])
        pltpu.async_copy(vals[b], nsh.at[rs[b]], sad[b], add=True)

    for b in range(_NBUF):
        issue_idx(b, b)
    for b in range(_NBUF - 1):
        wait_idx(b, b)
        issue_gather(b)

    def quad_body(k4, carry):
        for b in range(_NBUF):
            i = k4 * _NBUF + b

            @pl.when(i < _NCH)
            def _():
                wait_gather(b)
                process(i, b)

                @pl.when(i + _NBUF < _NCH)
                def _():
                    issue_idx(i + _NBUF, b)

                bn = (b + _NBUF - 1) % _NBUF

                @pl.when(i + _NBUF - 1 < _NCH)
                def _():
                    wait_idx(i + _NBUF - 1, bn)
                    issue_gather(bn)

        return carry

    lax.fori_loop(0, (_NCH + _NBUF - 1) // _NBUF, quad_body, 0)
    for j in range(_NBUF):
        i = _NCH - _NBUF + j
        if i >= 0:
            wait_write(i, i % _NBUF)
    plsc.subcore_barrier()

    @pl.when(jnp.logical_and(s == 0, c == 0))
    def _():
        pltpu.sync_copy(nsh, na_hbm)

    @pl.when(jnp.logical_and(s == 0, c == 1))
    def _():
        pltpu.sync_copy(nsh, nb_hbm)


_edge_attention = functools.partial(
    pl.kernel,
    out_type=[
        jax.ShapeDtypeStruct((_EPAD, _HEADS), jnp.float32),
        jax.ShapeDtypeStruct((_NPAD, _HPAD), jnp.float32),
        jax.ShapeDtypeStruct((_NPAD, _HPAD), jnp.float32),
    ],
    scratch_types=(
        [pltpu.VMEM((2, _CHUNK), jnp.int32)] * _NBUF +
        [pltpu.VMEM((_CHUNK, _EMB), jnp.float32)] * _NBUF +
        [pltpu.VMEM((_CHUNK, _EMB), jnp.float32)] * _NBUF +
        [pltpu.VMEM((_CHUNK, _HEADS), jnp.float32)] * _NBUF +
        [pltpu.VMEM((_CHUNK, _HPAD), jnp.float32)] * _NBUF +
        [pltpu.VMEM((_CHUNK,), jnp.int32)] * _NBUF +
        [pltpu.VMEM_SHARED((_NPAD, _HPAD), jnp.float32)] +
        [pltpu.SemaphoreType.DMA] * (5 * _NBUF)
    ),
    mesh=_mesh,
    compiler_params=_sc_params,
)(_edge_attention_body)


def _normalize_body(rows_hbm, exp_hbm, na_hbm, nb_hbm, out_hbm, *refs):
    rv = refs[0:_NBUF]
    ev = refs[_NBUF:2 * _NBUF]
    nav = refs[2 * _NBUF:3 * _NBUF]
    nbv = refs[3 * _NBUF:4 * _NBUF]
    av = refs[4 * _NBUF:5 * _NBUF]
    si = refs[5 * _NBUF:6 * _NBUF]
    sa = refs[6 * _NBUF:7 * _NBUF]
    sb = refs[7 * _NBUF:8 * _NBUF]
    sw = refs[8 * _NBUF:9 * _NBUF]
    c = lax.axis_index("c")
    s = lax.axis_index("s")
    wid = s * 2 + c
    base = wid * _PER_TILE

    def issue_idx(i, b):
        off = base + i * _CHUNK
        pltpu.async_copy(rows_hbm.at[pl.ds(off, _CHUNK)], rv[b], si[b])
        pltpu.async_copy(exp_hbm.at[pl.ds(off, _CHUNK)], ev[b], si[b])

    def wait_idx(i, b):
        off = base + i * _CHUNK
        pltpu.make_async_copy(rows_hbm.at[pl.ds(off, _CHUNK)], rv[b], si[b]).wait()
        pltpu.make_async_copy(exp_hbm.at[pl.ds(off, _CHUNK)], ev[b], si[b]).wait()

    def issue_gather(b):
        pltpu.async_copy(na_hbm.at[rv[b]], nav[b], sa[b])
        pltpu.async_copy(nb_hbm.at[rv[b]], nbv[b], sb[b])

    def wait_gather(b):
        pltpu.make_async_copy(na_hbm.at[rv[b]], nav[b], sa[b]).wait()
        pltpu.make_async_copy(nb_hbm.at[rv[b]], nbv[b], sb[b]).wait()

    def wait_write(i, b):
        off = base + i * _CHUNK
        pltpu.make_async_copy(av[b], out_hbm.at[pl.ds(off, _CHUNK)], sw[b]).wait()

    def process(i, b):
        @pl.when(i >= _NBUF)
        def _():
            wait_write(i - _NBUF, b)

        for g in range(_CHUNK // _L):
            ei = lax.iota(jnp.int32, _L) + (g * _L)
            acc = jnp.zeros((_L,), jnp.float32)
            for h in range(_HEADS):
                hs = jnp.full((_L,), h, jnp.int32)
                eh = plsc.load_gather(ev[b], [ei, hs])
                nh = (plsc.load_gather(nav[b], [ei, hs])
                      + plsc.load_gather(nbv[b], [ei, hs]))
                acc = acc + eh / (nh + 1e-8)
            av[b][pl.ds(g * _L, _L)] = acc
        off = base + i * _CHUNK
        pltpu.async_copy(av[b], out_hbm.at[pl.ds(off, _CHUNK)], sw[b])

    for b in range(_NBUF):
        issue_idx(b, b)
    for b in range(_NBUF - 1):
        wait_idx(b, b)
        issue_gather(b)

    def quad_body(k4, carry):
        for b in range(_NBUF):
            i = k4 * _NBUF + b

            @pl.when(i < _NCH)
            def _():
                wait_gather(b)
                process(i, b)

                @pl.when(i + _NBUF < _NCH)
                def _():
                    issue_idx(i + _NBUF, b)

                bn = (b + _NBUF - 1) % _NBUF

                @pl.when(i + _NBUF - 1 < _NCH)
                def _():
                    wait_idx(i + _NBUF - 1, bn)
                    issue_gather(bn)

        return carry

    lax.fori_loop(0, (_NCH + _NBUF - 1) // _NBUF, quad_body, 0)
    for j in range(_NBUF):
        i = _NCH - _NBUF + j
        if i >= 0:
            wait_write(i, i % _NBUF)


_normalize = functools.partial(
    pl.kernel,
    out_type=jax.ShapeDtypeStruct((_EPAD,), jnp.float32),
    scratch_types=(
        [pltpu.VMEM((_CHUNK,), jnp.int32)] * _NBUF +
        [pltpu.VMEM((_CHUNK, _HEADS), jnp.float32)] * _NBUF +
        [pltpu.VMEM((_CHUNK, _HPAD), jnp.float32)] * _NBUF +
        [pltpu.VMEM((_CHUNK, _HPAD), jnp.float32)] * _NBUF +
        [pltpu.VMEM((_CHUNK,), jnp.float32)] * _NBUF +
        [pltpu.SemaphoreType.DMA] * (4 * _NBUF)
    ),
    mesh=_mesh,
    compiler_params=_sc_params,
)(_normalize_body)


def kernel(embeds, edge_index, anchorset_id, dists_array, Wh, bh, qTrans,
           kTrans, vTrans):
    del vTrans
    f32 = jnp.float32
    set_emb = jnp.take(embeds, anchorset_id, axis=0)
    w1 = Wh[:_EMB]
    w2 = Wh[_EMB:]
    emb_p = jnp.pad(embeds, ((0, _NPAD - _N), (0, 0)))
    dst_p = jnp.pad(dists_array, ((0, _NPAD - _N), (0, 0)))
    q_tab, k_tab = pl.pallas_call(
        _qk_body,
        out_shape=[jax.ShapeDtypeStruct((_NPAD, _EMB), f32)] * 2,
    )(emb_p, dst_p, set_emb, w1, w2, bh.reshape(1, _EMB), qTrans, kTrans)

    rows = edge_index[0]
    cols = edge_index[1]
    ka, kb = jax.random.split(jax.random.key(1))
    tr = rows[jax.random.randint(ka, (_ADD,), 0, _E0)]
    tc = cols[jax.random.randint(kb, (_ADD,), 0, _E0)]
    loop = jnp.arange(_N, dtype=rows.dtype)
    new_rows = jnp.concatenate([tr, tc, loop, rows])
    new_cols = jnp.concatenate([tc, tr, loop, cols])
    rows_p = jnp.pad(new_rows, (0, _EPAD - _ETOT), constant_values=_N)
    cols_p = jnp.pad(new_cols, (0, _EPAD - _ETOT), constant_values=_N)
    rc = jnp.stack([rows_p.reshape(_NCHT, _CHUNK),
                    cols_p.reshape(_NCHT, _CHUNK)], axis=1)
    z = jnp.zeros((_NPAD, _HPAD), f32)

    exp_e, na, nb = _edge_attention(q_tab, k_tab, rc, z)
    att = _normalize(rows_p, exp_e, na, nb)
    return att[:_ETOT], new_rows, new_cols


# speedup vs baseline: 1.5532x; 1.0784x over previous
import functools

import jax
import jax.numpy as jnp
from jax import lax
from jax.experimental import pallas as pl
from jax.experimental.pallas import tpu as pltpu
from jax.experimental.pallas import tpu_sc as plsc

_N = 10000
_EMB = 32
_ANCH = 32
_HEADS = 4
_DH = 8
_E0 = 640000
_ADD = int(_E0 * 0.01)
_ETOT = 2 * _ADD + _N + _E0
_L = 16
_NW = 32
_CHUNK = 128
_NCH = -(-_ETOT // (_NW * _CHUNK))
_PER_TILE = _NCH * _CHUNK
_EPAD = _NW * _PER_TILE
_NCHT = _EPAD // _CHUNK
_NPAD = _N + 8
_HPAD = 8
_NBUF = 6


def _qk_body(emb_ref, dst_ref, se_ref, w1_ref, w2_ref, bh_ref, qt_ref,
             kt_ref, q_ref, k_ref):
    f32 = jnp.float32
    sw = jnp.dot(se_ref[...], w1_ref[...], preferred_element_type=f32)
    pos = (jnp.dot(dst_ref[...], sw, preferred_element_type=f32) * (1.0 / _ANCH)
           + jnp.dot(emb_ref[...], w2_ref[...], preferred_element_type=f32)
           + bh_ref[...])
    q_ref[...] = jnp.dot(pos, qt_ref[...], preferred_element_type=f32)
    k_ref[...] = jnp.dot(pos, kt_ref[...], preferred_element_type=f32)


_mesh = plsc.VectorSubcoreMesh(core_axis_name="c", subcore_axis_name="s")
_sc_params = pltpu.CompilerParams(
    needs_layout_passes=False, use_tc_tiling_on_sc=False)


def _edge_attention_body(q_hbm, k_hbm, rc_hbm, z_hbm,
                         exp_hbm, na_hbm, nb_hbm,
                         *refs):
    rcv = refs[0:_NBUF]
    qv = refs[_NBUF:2 * _NBUF]
    kv = refs[2 * _NBUF:3 * _NBUF]
    ev = refs[3 * _NBUF:4 * _NBUF]
    vals = refs[4 * _NBUF:5 * _NBUF]
    rs = refs[5 * _NBUF:6 * _NBUF]
    nsh = refs[6 * _NBUF]
    si = refs[6 * _NBUF + 1:7 * _NBUF + 1]
    sq = refs[7 * _NBUF + 1:8 * _NBUF + 1]
    sk = refs[8 * _NBUF + 1:9 * _NBUF + 1]
    sew = refs[9 * _NBUF + 1:10 * _NBUF + 1]
    sad = refs[10 * _NBUF + 1:11 * _NBUF + 1]
    c = lax.axis_index("c")
    s = lax.axis_index("s")
    wid = s * 2 + c
    base = wid * _PER_TILE
    chbase = wid * _NCH
    for b in range(_NBUF):
        pltpu.sync_copy(z_hbm.at[pl.ds(0, _CHUNK)], vals[b])

    @pl.when(s == 0)
    def _():
        pltpu.sync_copy(z_hbm, nsh)

    plsc.subcore_barrier()

    def issue_idx(i, b):
        pltpu.async_copy(rc_hbm.at[chbase + i], rcv[b], si[b])

    def wait_idx(i, b):
        pltpu.make_async_copy(rc_hbm.at[chbase + i], rcv[b], si[b]).wait()

    def issue_gather(b):
        pltpu.async_copy(q_hbm.at[rcv[b].at[0]], qv[b], sq[b])
        pltpu.async_copy(k_hbm.at[rcv[b].at[1]], kv[b], sk[b])

    def wait_gather(b):
        pltpu.make_async_copy(q_hbm.at[rcv[b].at[0]], qv[b], sq[b]).wait()
        pltpu.make_async_copy(k_hbm.at[rcv[b].at[1]], kv[b], sk[b]).wait()

    def wait_write(i, b):
        off = base + i * _CHUNK
        pltpu.make_async_copy(ev[b], exp_hbm.at[pl.ds(off, _CHUNK)], sew[b]).wait()
        pltpu.make_async_copy(vals[b], nsh.at[rs[b]], sad[b]).wait()

    def process(i, b):
        @pl.when(i >= _NBUF)
        def _():
            wait_write(i - _NBUF, b)

        for g in range(_CHUNK // _L):
            ei = lax.iota(jnp.int32, _L) + (g * _L)
            for h in range(_HEADS):
                acc = None
                for d in range(_DH):
                    ci = jnp.full((_L,), h * _DH + d, jnp.int32)
                    qc = plsc.load_gather(qv[b], [ei, ci])
                    kc = plsc.load_gather(kv[b], [ei, ci])
                    acc = qc * kc if acc is None else acc + qc * kc
                att = jnp.minimum(jnp.maximum(acc, -10.0), 10.0)
                ex = jnp.exp(att)
                hs = jnp.full((_L,), h, jnp.int32)
                plsc.store_scatter(vals[b], [ei, hs], ex)
                plsc.store_scatter(ev[b], [ei, hs], ex)
        for j in range(_CHUNK // _L):
            ix = pl.ds(j * _L, _L)
            rs[b][ix] = rcv[b][0, ix]
        off = base + i * _CHUNK
        pltpu.async_copy(ev[b], exp_hbm.at[pl.ds(off, _CHUNK)], sew[b])
        pltpu.async_copy(vals[b], nsh.at[rs[b]], sad[b], add=True)

    for b in range(_NBUF):
        issue_idx(b, b)
    for b in range(_NBUF - 1):
        wait_idx(b, b)
        issue_gather(b)

    def quad_body(k4, carry):
        for b in range(_NBUF):
            i = k4 * _NBUF + b

            @pl.when(i < _NCH)
            def _():
                wait_gather(b)
                process(i, b)

                @pl.when(i + _NBUF < _NCH)
                def _():
                    issue_idx(i + _NBUF, b)

                bn = (b + _NBUF - 1) % _NBUF

                @pl.when(i + _NBUF - 1 < _NCH)
                def _():
                    wait_idx(i + _NBUF - 1, bn)
                    issue_gather(bn)

        return carry

    lax.fori_loop(0, (_NCH + _NBUF - 1) // _NBUF, quad_body, 0)
    for j in range(_NBUF):
        i = _NCH - _NBUF + j
        if i >= 0:
            wait_write(i, i % _NBUF)
    plsc.subcore_barrier()

    @pl.when(jnp.logical_and(s == 0, c == 0))
    def _():
        pltpu.sync_copy(nsh, na_hbm)

    @pl.when(jnp.logical_and(s == 0, c == 1))
    def _():
        pltpu.sync_copy(nsh, nb_hbm)


_edge_attention = functools.partial(
    pl.kernel,
    out_type=[
        jax.ShapeDtypeStruct((_EPAD, _HEADS), jnp.float32),
        jax.ShapeDtypeStruct((_NPAD, _HPAD), jnp.float32),
        jax.ShapeDtypeStruct((_NPAD, _HPAD), jnp.float32),
    ],
    scratch_types=(
        [pltpu.VMEM((2, _CHUNK), jnp.int32)] * _NBUF +
        [pltpu.VMEM((_CHUNK, _EMB), jnp.float32)] * _NBUF +
        [pltpu.VMEM((_CHUNK, _EMB), jnp.float32)] * _NBUF +
        [pltpu.VMEM((_CHUNK, _HEADS), jnp.float32)] * _NBUF +
        [pltpu.VMEM((_CHUNK, _HPAD), jnp.float32)] * _NBUF +
        [pltpu.VMEM((_CHUNK,), jnp.int32)] * _NBUF +
        [pltpu.VMEM_SHARED((_NPAD, _HPAD), jnp.float32)] +
        [pltpu.SemaphoreType.DMA] * (5 * _NBUF)
    ),
    mesh=_mesh,
    compiler_params=_sc_params,
)(_edge_attention_body)


def _normalize_body(rows_hbm, exp_hbm, na_hbm, nb_hbm, out_hbm, *refs):
    rv = refs[0:_NBUF]
    ev = refs[_NBUF:2 * _NBUF]
    nav = refs[2 * _NBUF:3 * _NBUF]
    nbv = refs[3 * _NBUF:4 * _NBUF]
    av = refs[4 * _NBUF:5 * _NBUF]
    si = refs[5 * _NBUF:6 * _NBUF]
    sa = refs[6 * _NBUF:7 * _NBUF]
    sb = refs[7 * _NBUF:8 * _NBUF]
    sw = refs[8 * _NBUF:9 * _NBUF]
    c = lax.axis_index("c")
    s = lax.axis_index("s")
    wid = s * 2 + c
    base = wid * _PER_TILE

    def issue_idx(i, b):
        off = base + i * _CHUNK
        pltpu.async_copy(rows_hbm.at[pl.ds(off, _CHUNK)], rv[b], si[b])
        pltpu.async_copy(exp_hbm.at[pl.ds(off, _CHUNK)], ev[b], si[b])

    def wait_idx(i, b):
        off = base + i * _CHUNK
        pltpu.make_async_copy(rows_hbm.at[pl.ds(off, _CHUNK)], rv[b], si[b]).wait()
        pltpu.make_async_copy(exp_hbm.at[pl.ds(off, _CHUNK)], ev[b], si[b]).wait()

    def issue_gather(b):
        pltpu.async_copy(na_hbm.at[rv[b]], nav[b], sa[b])
        pltpu.async_copy(nb_hbm.at[rv[b]], nbv[b], sb[b])

    def wait_gather(b):
        pltpu.make_async_copy(na_hbm.at[rv[b]], nav[b], sa[b]).wait()
        pltpu.make_async_copy(nb_hbm.at[rv[b]], nbv[b], sb[b]).wait()

    def wait_write(i, b):
        off = base + i * _CHUNK
        pltpu.make_async_copy(av[b], out_hbm.at[pl.ds(off, _CHUNK)], sw[b]).wait()

    def process(i, b):
        @pl.when(i >= _NBUF)
        def _():
            wait_write(i - _NBUF, b)

        for g in range(_CHUNK // _L):
            ei = lax.iota(jnp.int32, _L) + (g * _L)
            acc = jnp.zeros((_L,), jnp.float32)
            for h in range(_HEADS):
                hs = jnp.full((_L,), h, jnp.int32)
                eh = plsc.load_gather(ev[b], [ei, hs])
                nh = (plsc.load_gather(nav[b], [ei, hs])
                      + plsc.load_gather(nbv[b], [ei, hs]))
                acc = acc + eh / (nh + 1e-8)
            av[b][pl.ds(g * _L, _L)] = acc
        off = base + i * _CHUNK
        pltpu.async_copy(av[b], out_hbm.at[pl.ds(off, _CHUNK)], sw[b])

    for b in range(_NBUF):
        issue_idx(b, b)
    for b in range(_NBUF - 1):
        wait_idx(b, b)
        issue_gather(b)

    def quad_body(k4, carry):
        for b in range(_NBUF):
            i = k4 * _NBUF + b

            @pl.when(i < _NCH)
            def _():
                wait_gather(b)
                process(i, b)

                @pl.when(i + _NBUF < _NCH)
                def _():
                    issue_idx(i + _NBUF, b)

                bn = (b + _NBUF - 1) % _NBUF

                @pl.when(i + _NBUF - 1 < _NCH)
                def _():
                    wait_idx(i + _NBUF - 1, bn)
                    issue_gather(bn)

        return carry

    lax.fori_loop(0, (_NCH + _NBUF - 1) // _NBUF, quad_body, 0)
    for j in range(_NBUF):
        i = _NCH - _NBUF + j
        if i >= 0:
            wait_write(i, i % _NBUF)


_normalize = functools.partial(
    pl.kernel,
    out_type=jax.ShapeDtypeStruct((_EPAD,), jnp.float32),
    scratch_types=(
        [pltpu.VMEM((_CHUNK,), jnp.int32)] * _NBUF +
        [pltpu.VMEM((_CHUNK, _HEADS), jnp.float32)] * _NBUF +
        [pltpu.VMEM((_CHUNK, _HPAD), jnp.float32)] * _NBUF +
        [pltpu.VMEM((_CHUNK, _HPAD), jnp.float32)] * _NBUF +
        [pltpu.VMEM((_CHUNK,), jnp.float32)] * _NBUF +
        [pltpu.SemaphoreType.DMA] * (4 * _NBUF)
    ),
    mesh=_mesh,
    compiler_params=_sc_params,
)(_normalize_body)


def kernel(embeds, edge_index, anchorset_id, dists_array, Wh, bh, qTrans,
           kTrans, vTrans):
    del vTrans
    f32 = jnp.float32
    set_emb = jnp.take(embeds, anchorset_id, axis=0)
    w1 = Wh[:_EMB]
    w2 = Wh[_EMB:]
    emb_p = jnp.pad(embeds, ((0, _NPAD - _N), (0, 0)))
    dst_p = jnp.pad(dists_array, ((0, _NPAD - _N), (0, 0)))
    q_tab, k_tab = pl.pallas_call(
        _qk_body,
        out_shape=[jax.ShapeDtypeStruct((_NPAD, _EMB), f32)] * 2,
    )(emb_p, dst_p, set_emb, w1, w2, bh.reshape(1, _EMB), qTrans, kTrans)

    rows = edge_index[0]
    cols = edge_index[1]
    ka, kb = jax.random.split(jax.random.key(1))
    tr = rows[jax.random.randint(ka, (_ADD,), 0, _E0)]
    tc = cols[jax.random.randint(kb, (_ADD,), 0, _E0)]
    loop = jnp.arange(_N, dtype=rows.dtype)
    new_rows = jnp.concatenate([tr, tc, loop, rows])
    new_cols = jnp.concatenate([tc, tr, loop, cols])
    rows_p = jnp.pad(new_rows, (0, _EPAD - _ETOT), constant_values=_N)
    cols_p = jnp.pad(new_cols, (0, _EPAD - _ETOT), constant_values=_N)
    rc = jnp.stack([rows_p.reshape(_NCHT, _CHUNK),
                    cols_p.reshape(_NCHT, _CHUNK)], axis=1)
    z = jnp.zeros((_NPAD, _HPAD), f32)

    exp_e, na, nb = _edge_attention(q_tab, k_tab, rc, z)
    att = _normalize(rows_p, exp_e, na, nb)
    return att[:_ETOT], new_rows, new_cols
